# baseline (speedup 1.0000x reference)
import functools

import jax
import jax.numpy as jnp
from jax import lax
from jax.experimental import pallas as pl
from jax.experimental.pallas import tpu as pltpu, tpu_sc as plsc

M, P, D = 16, 1024, 768
NC, NS, L = 2, 16, 16
NW = NC * NS
PPW = P // NW
SL = D // L
GRP = 8
NB = SL // GRP

_mesh = plsc.VectorSubcoreMesh(core_axis_name="c", subcore_axis_name="s")


@functools.partial(
    pl.kernel,
    out_type=jax.ShapeDtypeStruct((M, P, D), jnp.float32),
    mesh=_mesh,
    scratch_types=[
        pltpu.VMEM((PPW, D), jnp.float32),
        pltpu.VMEM((PPW, D), jnp.float32),
        pltpu.VMEM((PPW, M), jnp.float32),
        pltpu.VMEM((PPW, D), jnp.float32),
        pltpu.VMEM((PPW, D), jnp.float32),
        pltpu.SemaphoreType.DMA,
        pltpu.SemaphoreType.DMA,
        pltpu.SemaphoreType.DMA,
    ],
)
def _sc_kernel(feat_hbm, pos_hbm, maskT_hbm, out_hbm,
               a_v, b_v, mask_v, ob0_v, ob1_v, sem0, sem1, sem_in):
    wid = lax.axis_index("s") * NC + lax.axis_index("c")
    base = wid * PPW
    cp_a = pltpu.async_copy(feat_hbm.at[pl.ds(base, PPW)], a_v, sem0)
    cp_b = pltpu.async_copy(pos_hbm.at[pl.ds(base, PPW)], b_v, sem1)
    cp_m = pltpu.async_copy(maskT_hbm.at[pl.ds(base, PPW)], mask_v, sem_in)
    cp_a.wait()
    cp_b.wait()

    def add_blk(i, carry):
        p = i // NB
        j0 = lax.rem(i, NB) * GRP
        for j in range(GRP):
            sl = pl.ds(j0 * L + j * L, L)
            a_v[p, sl] = a_v[p, sl] + b_v[p, sl]
        return carry

    lax.fori_loop(0, PPW * NB, add_blk, 0)
    cp_m.wait()

    obufs = (ob0_v, ob1_v)
    sems = (sem0, sem1)

    def scale_rows(m, ob):
        def blk(i, c):
            p = i // NB
            j0 = lax.rem(i, NB) * GRP
            mval = mask_v[p, :][m]
            for j in range(GRP):
                sl = pl.ds(j0 * L + j * L, L)
                ob[p, sl] = a_v[p, sl] * mval
            return c

        lax.fori_loop(0, PPW * NB, blk, 0)

    def out_dma(m, ob, sem):
        return pltpu.async_copy(ob, out_hbm.at[m, pl.ds(base, PPW)], sem)

    for m in range(M):
        ob, sem = obufs[m % 2], sems[m % 2]
        if m >= 2:
            pltpu.make_async_copy(ob, out_hbm.at[m, pl.ds(base, PPW)], sem).wait()
        scale_rows(m, ob)
        out_dma(m, ob, sem)

    pltpu.make_async_copy(ob0_v, out_hbm.at[0, pl.ds(base, PPW)], sem0).wait()
    pltpu.make_async_copy(ob1_v, out_hbm.at[1, pl.ds(base, PPW)], sem1).wait()


def kernel(image_features, pos_table, masks):
    maskT = masks.T.astype(jnp.float32)
    return _sc_kernel(image_features, pos_table, maskT)


# speedup vs baseline: 1.3717x; 1.3717x over previous
import functools

import jax
import jax.numpy as jnp
from jax import lax
from jax.experimental import pallas as pl
from jax.experimental.pallas import tpu as pltpu, tpu_sc as plsc

M, P, D = 16, 1024, 768
NC, NS, L = 2, 16, 16
NW = NC * NS
PPW = P // NW
SL = D // L
NBUF = 3

_mesh = plsc.VectorSubcoreMesh(core_axis_name="c", subcore_axis_name="s")


@functools.partial(
    pl.kernel,
    out_type=jax.ShapeDtypeStruct((M, P, D), jnp.float32),
    mesh=_mesh,
    scratch_types=[
        pltpu.VMEM((PPW, D), jnp.float32),
        pltpu.VMEM((PPW, D), jnp.float32),
        pltpu.VMEM((PPW, M), jnp.float32),
        pltpu.SMEM((PPW * M,), jnp.float32),
        pltpu.VMEM((PPW, D), jnp.float32),
        pltpu.VMEM((PPW, D), jnp.float32),
        pltpu.VMEM((PPW, D), jnp.float32),
        pltpu.SemaphoreType.DMA,
        pltpu.SemaphoreType.DMA,
        pltpu.SemaphoreType.DMA,
        pltpu.SemaphoreType.DMA,
    ],
)
def _sc_kernel(feat_hbm, pos_hbm, maskT_hbm, out_hbm,
               a_v, b_v, mask_v, mask_s, ob0_v, ob1_v, ob2_v,
               sem0, sem1, sem2, sem_in):
    wid = lax.axis_index("s") * NC + lax.axis_index("c")
    base = wid * PPW
    cp_a = pltpu.async_copy(feat_hbm.at[pl.ds(base, PPW)], a_v, sem0)
    cp_b = pltpu.async_copy(pos_hbm.at[pl.ds(base, PPW)], b_v, sem1)
    cp_m = pltpu.async_copy(maskT_hbm.at[pl.ds(base, PPW)], mask_v, sem_in)
    cp_a.wait()
    cp_b.wait()
    cp_m.wait()

    obufs = (ob0_v, ob1_v, ob2_v)
    sems = (sem0, sem1, sem2)

    def scale_rows(m, ob):
        def row(p, c):
            mval = mask_s[p * M + m]
            for j in range(SL):
                sl = pl.ds(j * L, L)
                ob[p, sl] = a_v[p, sl] * mval
            return c

        lax.fori_loop(0, PPW, row, 0)

    def row0(p, carry):
        vec = mask_v[p, :]
        for mm in range(M):
            mask_s[p * M + mm] = vec[mm]
        mval = vec[0]
        for j in range(SL):
            sl = pl.ds(j * L, L)
            f = a_v[p, sl] + b_v[p, sl]
            a_v[p, sl] = f
            ob0_v[p, sl] = f * mval
        return carry

    lax.fori_loop(0, PPW, row0, 0)
    pltpu.async_copy(ob0_v, out_hbm.at[0, pl.ds(base, PPW)], sem0)

    for k in range(1, NBUF):
        scale_rows(k, obufs[k])
        pltpu.async_copy(obufs[k], out_hbm.at[k, pl.ds(base, PPW)], sems[k])

    def group(i, carry):
        m0 = NBUF + i * NBUF
        for k in range(NBUF):
            ob, sem = obufs[k], sems[k]
            pltpu.make_async_copy(ob, out_hbm.at[m0 + k, pl.ds(base, PPW)], sem).wait()
            scale_rows(m0 + k, ob)
            pltpu.async_copy(ob, out_hbm.at[m0 + k, pl.ds(base, PPW)], sem)
        return carry

    lax.fori_loop(0, (M - NBUF) // NBUF, group, 0)

    m_last = M - 1
    pltpu.make_async_copy(ob0_v, out_hbm.at[m_last, pl.ds(base, PPW)], sem0).wait()
    scale_rows(m_last, ob0_v)
    pltpu.async_copy(ob0_v, out_hbm.at[m_last, pl.ds(base, PPW)], sem0)

    for k in range(NBUF):
        pltpu.make_async_copy(obufs[k], out_hbm.at[k, pl.ds(base, PPW)], sems[k]).wait()


def kernel(image_features, pos_table, masks):
    maskT = masks.T.astype(jnp.float32)
    return _sc_kernel(image_features, pos_table, maskT)
